# baseline (speedup 1.0000x reference)
import jax
import jax.numpy as jnp
from jax.experimental import pallas as pl
from jax.experimental.pallas import tpu as pltpu

MASK_VALUE = -10000000.0

B = 4096
S = 32
H = 50
E = 32
BLK = 256


def _attn_body(u_ref, item_ref, nei_ref, rel_ref, hist_ref, nmask_ref,
               hmask_ref, wu_ref, bu_ref, out_ref):
    u = u_ref[...]
    q = jnp.tanh(jnp.dot(u, wu_ref[...],
                         preferred_element_type=jnp.float32) + bu_ref[...])
    item_e = item_ref[...]

    nei = nei_ref[...]
    rel = rel_ref[...]
    scores = jnp.sum(q[:, None, :] * (nei + rel), axis=-1) + nmask_ref[...]
    scores = scores - jnp.max(scores, axis=-1, keepdims=True)
    w = jnp.exp(scores)
    w = w / jnp.sum(w, axis=-1, keepdims=True)
    local_ctx = jnp.sum(w[:, :, None] * nei, axis=1)

    hist = hist_ref[...]
    hscores = jnp.sum(item_e[:, None, :] * hist, axis=-1) + hmask_ref[...]
    hscores = hscores - jnp.max(hscores, axis=-1, keepdims=True)
    hw = jnp.exp(hscores)
    hw = hw / jnp.sum(hw, axis=-1, keepdims=True)
    user_ctx = jnp.sum(hw[:, :, None] * hist, axis=1)

    user_repr = q + user_ctx
    item_repr = item_e + local_ctx
    out_ref[...] = jnp.sum(user_repr * item_repr, axis=-1)


def _attention(u, item_e, nei, rel, hist, nmask, hmask, W_u, b_u):
    grid = (B // BLK,)
    return pl.pallas_call(
        _attn_body,
        grid=grid,
        in_specs=[
            pl.BlockSpec((BLK, E), lambda i: (i, 0)),
            pl.BlockSpec((BLK, E), lambda i: (i, 0)),
            pl.BlockSpec((BLK, S, E), lambda i: (i, 0, 0)),
            pl.BlockSpec((BLK, S, E), lambda i: (i, 0, 0)),
            pl.BlockSpec((BLK, H, E), lambda i: (i, 0, 0)),
            pl.BlockSpec((BLK, S), lambda i: (i, 0)),
            pl.BlockSpec((BLK, H), lambda i: (i, 0)),
            pl.BlockSpec((E, E), lambda i: (0, 0)),
            pl.BlockSpec((1, E), lambda i: (0, 0)),
        ],
        out_specs=pl.BlockSpec((BLK,), lambda i: (i,)),
        out_shape=jax.ShapeDtypeStruct((B,), jnp.float32),
    )(u, item_e, nei, rel, hist, nmask, hmask, W_u, b_u)


def kernel(user_ids, item_ids, neighbour_ids, relation_ids, neighbour_masks,
           interacted_item_ids, interacted_item_masks,
           user_table, entity_table, relation_table, W_u, b_u):
    u = jnp.take(user_table, user_ids, axis=0)
    item_e = jnp.take(entity_table, item_ids, axis=0)
    nei = jnp.take(entity_table, neighbour_ids, axis=0)
    rel = jnp.take(relation_table, relation_ids, axis=0)
    hist = jnp.take(entity_table, interacted_item_ids, axis=0)
    nmask = (~neighbour_masks).astype(jnp.float32) * MASK_VALUE
    hmask = (~interacted_item_masks).astype(jnp.float32) * MASK_VALUE
    return _attention(u, item_e, nei, rel, hist, nmask, hmask,
                      W_u, b_u.reshape(1, E))
